# baseline (speedup 1.0000x reference)
import functools

import jax
import jax.numpy as jnp
from jax import lax
from jax.experimental import pallas as pl
from jax.experimental.pallas import tpu as pltpu
from jax.experimental.pallas import tpu_sc as plsc

K = 5
ROWS = 128
VOCAB = 100000
TILE = 128
CHUNK_T = 39
CW = CHUNK_T * TILE
NCHUNK = 10
NPAIR = NCHUNK // 2
HALF_COLS = NCHUNK * CW
REM_START = 2 * HALF_COLS
REM = VOCAB - REM_START
BIG = 1e9


def _sc_count_kernel():
    info = plsc.get_sparse_core_info()
    nc, ns = info.num_cores, info.num_subcores
    nw = nc * ns

    mesh = plsc.VectorSubcoreMesh(core_axis_name="c", subcore_axis_name="s")

    @functools.partial(
        pl.kernel,
        mesh=mesh,
        compiler_params=pltpu.CompilerParams(needs_layout_passes=False),
        out_type=jax.ShapeDtypeStruct((nw, 128), jnp.float32),
        scratch_types=[
            pltpu.VMEM((ROWS,), jnp.int32),
            pltpu.VMEM((8, 8, TILE), jnp.float32),
            pltpu.VMEM((8, CW), jnp.float32),
            pltpu.VMEM((8, CW), jnp.float32),
            pltpu.VMEM((8, REM), jnp.float32),
            pltpu.VMEM((128,), jnp.float32),
            pltpu.SemaphoreType.DMA,
            pltpu.SemaphoreType.DMA,
            pltpu.SemaphoreType.DMA,
        ],
    )
    def sc_k(y_hbm, lbl_hbm, out_hbm, lbl_v, vstage, buf_a, buf_b, rbuf,
             ostage, sem_a, sem_b, sem_v):
        wid = lax.axis_index("s") * nc + lax.axis_index("c")
        slab = wid >> 1
        chalf = wid & 1
        row0 = pl.multiple_of(slab * 8, 8)
        lanes = lax.iota(jnp.int32, 16)
        bufs = (buf_a, buf_b)
        sems = (sem_a, sem_b)
        col_base = pl.multiple_of(chalf * HALF_COLS, TILE)

        pltpu.sync_copy(lbl_hbm, lbl_v)

        def start(g, par):
            return pltpu.async_copy(
                y_hbm.at[pl.ds(row0, 8), pl.ds(col_base + g * CW, CW)],
                bufs[par], sems[par])

        start(0, 0)
        rem_handle = pltpu.async_copy(
            y_hbm.at[pl.ds(row0, 8), pl.ds(REM_START, REM)], rbuf, sem_v)

        lab_b, l_s, vcopies = [], [], []
        for rr in range(8):
            lb = plsc.load_gather(
                lbl_v, [jnp.full((16,), row0 + rr, jnp.int32)])
            lab_b.append(lb)
            ls = jnp.max(lb)
            l_s.append(ls)
            ltile = pl.multiple_of((ls >> 7) << 7, TILE)
            vcopies.append(pltpu.async_copy(
                y_hbm.at[pl.ds(row0, 8), pl.ds(ltile, TILE)],
                vstage.at[rr], sem_v))
        for c in vcopies:
            c.wait()

        v_b, p_b = [], []
        for rr in range(8):
            rsel = jnp.full((16,), rr, jnp.int32)
            vb = plsc.load_gather(
                vstage, [rsel, rsel, jnp.full((16,), 1, jnp.int32) * (l_s[rr] & 127)])
            v_b.append(vb)
            b = plsc.bitcast(vb, jnp.int32)
            minus_tiny = jnp.full((16,), -2147483647, jnp.int32)
            pb = jnp.where(b > 0, b - 1,
                           jnp.where((b << 1) == 0, minus_tiny, b + 1))
            p_b.append(plsc.bitcast(pb, jnp.float32))

        accs = [jnp.zeros((16,), jnp.int32) for _ in range(8)]

        def count_tile_cols(buf, g, t, accs8, scale=None):
            accs8 = list(accs8)
            c0 = col_base + g * CW + t * TILE
            for j in range(8):
                col_j = c0 + j * 16 + lanes
                for rr in range(8):
                    x = buf[rr, pl.ds(t * TILE + j * 16, 16)]
                    thr = jnp.where(col_j < lab_b[rr], p_b[rr], v_b[rr])
                    c = plsc.all_reduce_population_count(x > thr)
                    if scale is not None:
                        c = c * scale
                    accs8[rr] = accs8[rr] + c
            return tuple(accs8)

        def wait(which):
            pltpu.make_async_copy(
                y_hbm.at[pl.ds(row0, 8), pl.ds(0, CW)],
                bufs[which], sems[which]).wait()

        def chunk_phase(g, buf_par, accs8):
            wait(buf_par)

            def tbody(t, a):
                return count_tile_cols(bufs[buf_par], g, t, a)

            accs8 = lax.fori_loop(0, CHUNK_T, tbody, accs8)
            return accs8

        start(1, 1)

        def pair_body(p, accs8):
            g0 = 2 * p
            accs8 = chunk_phase(g0, 0, accs8)

            @pl.when(p < NPAIR - 1)
            def _():
                start(g0 + 2, 0)

            accs8 = chunk_phase(g0 + 1, 1, accs8)

            @pl.when(p < NPAIR - 1)
            def _():
                start(g0 + 3, 1)

            return accs8

        accs = list(lax.fori_loop(0, NPAIR, pair_body, tuple(accs)))

        rem_handle.wait()
        rem_on = jnp.where(chalf == 0, 1, 0).astype(jnp.int32)

        def rem_body(j, accs8):
            accs8 = list(accs8)
            col_j = REM_START + j * 16 + lanes
            for rr in range(8):
                x = rbuf[rr, pl.ds(j * 16, 16)]
                thr = jnp.where(col_j < lab_b[rr], p_b[rr], v_b[rr])
                accs8[rr] = accs8[rr] + rem_on * plsc.all_reduce_population_count(
                    x > thr)
            return tuple(accs8)

        accs = list(lax.fori_loop(0, REM // 16, rem_body, tuple(accs)))

        rankvec = jnp.full((16,), BIG, jnp.float32)
        for rr in range(8):
            s = jnp.sum(accs[rr]).astype(jnp.float32)
            rankvec = jnp.where(lanes == rr, jnp.full((16,), s, jnp.float32),
                                rankvec)
        big16 = jnp.full((16,), BIG, jnp.float32)
        for j in range(8):
            ostage[pl.ds(j * 16, 16)] = rankvec if j == 0 else big16
        pltpu.sync_copy(ostage, out_hbm.at[wid])

    return sc_k, nw


def _tc_reduce(partials):
    def body(x_ref, o_ref):
        x = x_ref[...]
        pairs = jnp.reshape(x, (16, 2, 128))
        ranks16 = pairs[:, 0, :] + pairs[:, 1, :]
        correct = (ranks16 < jnp.float32(16 * K)).astype(jnp.float32)
        o_ref[...] = jnp.sum(correct, axis=(0, 1), keepdims=True) * jnp.float32(
            1.0 / ROWS
        )

    return pl.pallas_call(
        body,
        out_shape=jax.ShapeDtypeStruct((1, 1), jnp.float32),
    )(partials)


def kernel(y_probs, y_true_label):
    labels = y_true_label.astype(jnp.int32)
    sc_k, nw = _sc_count_kernel()
    partials = sc_k(y_probs, labels)
    return _tc_reduce(partials)[0, 0]


# speedup vs baseline: 3.3082x; 3.3082x over previous
import functools

import jax
import jax.numpy as jnp
from jax import lax
from jax.experimental import pallas as pl
from jax.experimental.pallas import tpu as pltpu
from jax.experimental.pallas import tpu_sc as plsc

K = 5
ROWS = 128
VOCAB = 100000
VT = 8
CHUNK_V = 312
NCHUNK = 10
NPAIR = NCHUNK // 2
SLICE_V = NCHUNK * CHUNK_V
REM_START = 32 * SLICE_V
REM = VOCAB - REM_START


def _sc_count_kernel():
    info = plsc.get_sparse_core_info()
    nc, ns = info.num_cores, info.num_subcores
    nw = nc * ns

    mesh = plsc.VectorSubcoreMesh(core_axis_name="c", subcore_axis_name="s")

    @functools.partial(
        pl.kernel,
        mesh=mesh,
        compiler_params=pltpu.CompilerParams(needs_layout_passes=False),
        out_type=jax.ShapeDtypeStruct((nw, 128), jnp.float32),
        scratch_types=[
            pltpu.VMEM((ROWS,), jnp.int32),
            pltpu.VMEM((8, VT, ROWS), jnp.float32),
            pltpu.VMEM((16, 16), jnp.float32),
            pltpu.VMEM((CHUNK_V, ROWS), jnp.float32),
            pltpu.VMEM((CHUNK_V, ROWS), jnp.float32),
            pltpu.VMEM((REM, ROWS), jnp.float32),
            pltpu.VMEM((16,), jnp.float32),
            pltpu.VMEM((128,), jnp.float32),
            pltpu.VMEM_SHARED((16, 16), jnp.float32),
            pltpu.SemaphoreType.DMA,
            pltpu.SemaphoreType.DMA,
            pltpu.SemaphoreType.DMA,
        ],
    )
    def sc_k(y_hbm, lbl_hbm, out_hbm, lbl_v, vstage, vloc, buf_a, buf_b,
             rbuf, vslot, ostage, vshared, sem_a, sem_b, sem_v):
        sid = lax.axis_index("s")
        cid = lax.axis_index("c")
        wid = sid * nc + cid
        lanes = lax.iota(jnp.int32, 16)
        bufs = (buf_a, buf_b)
        sems = (sem_a, sem_b)
        j0 = pl.multiple_of(wid * SLICE_V, VT)

        def start(g, par):
            return pltpu.async_copy(
                y_hbm.at[pl.ds(j0 + g * CHUNK_V, CHUNK_V), :],
                bufs[par], sems[par])

        start(0, 0)
        start(1, 1)
        rem_handle = pltpu.async_copy(
            y_hbm.at[pl.ds(REM_START, REM), :], rbuf, sem_v)

        pltpu.sync_copy(lbl_hbm, lbl_v)

        lab_s, copies = [], []
        for i in range(VT):
            lb = plsc.load_gather(
                lbl_v, [jnp.full((16,), 1, jnp.int32) * (sid * VT + i)])
            ls = jnp.max(lb)
            lab_s.append(ls)
            jt = pl.multiple_of((ls >> 3) << 3, VT)
            copies.append(pltpu.async_copy(
                y_hbm.at[pl.ds(jt, VT), :], vstage.at[i], sem_v))
        for c in copies:
            c.wait()

        base_lane = (sid * VT) & 15
        myv = jnp.zeros((16,), jnp.float32)
        for i in range(VT):
            ls = lab_s[i]
            b = sid * VT + i
            vv = plsc.load_gather(
                vstage,
                [jnp.full((16,), i, jnp.int32),
                 jnp.full((16,), 1, jnp.int32) * (ls & (VT - 1)),
                 jnp.full((16,), 1, jnp.int32) * b])
            myv = jnp.where(lanes == base_lane + i, vv, myv)
        vslot[...] = myv
        pltpu.sync_copy(vslot, vshared.at[sid])
        plsc.subcore_barrier()
        pltpu.sync_copy(vshared, vloc)

        v_g, p_g, lab_g = [], [], []
        minus_tiny = jnp.full((16,), -2147483647, jnp.int32)
        for g in range(8):
            va = vloc[2 * g, :] + vloc[2 * g + 1, :]
            v_g.append(va)
            bb = plsc.bitcast(va, jnp.int32)
            pb = jnp.where(bb > 0, bb - 1,
                           jnp.where((bb << 1) == 0, minus_tiny, bb + 1))
            p_g.append(plsc.bitcast(pb, jnp.float32))
            lab_g.append(lbl_v[pl.ds(16 * g, 16)])

        accs = [jnp.zeros((16,), jnp.int32) for _ in range(8)]
        one = jnp.full((16,), 1, jnp.int32)
        zero = jnp.zeros((16,), jnp.int32)

        def lane_group_chunk(buf, jbase, nt, g, acc):
            vb, pb, lb = v_g[g], p_g[g], lab_g[g]
            jb0 = jnp.full((16,), 1, jnp.int32) * jbase

            def tbody(tt, a):
                jv0 = jb0 + tt * VT
                for jj in range(VT):
                    x = buf[tt * VT + jj, pl.ds(16 * g, 16)]
                    thr = jnp.where(jv0 + jj < lb, pb, vb)
                    a = a + jnp.where(x > thr, one, zero)
                return a

            return lax.fori_loop(0, nt, tbody, acc)

        def chunk_phase(g_chunk, par, accs8):
            pltpu.make_async_copy(
                y_hbm.at[pl.ds(0, CHUNK_V), :], bufs[par], sems[par]).wait()
            accs8 = list(accs8)
            jbase = j0 + g_chunk * CHUNK_V
            for g in range(8):
                accs8[g] = lane_group_chunk(bufs[par], jbase,
                                            CHUNK_V // VT, g, accs8[g])
            return tuple(accs8)

        def pair_body(p, accs8):
            g0 = 2 * p
            accs8 = chunk_phase(g0, 0, accs8)

            @pl.when(p < NPAIR - 1)
            def _():
                start(g0 + 2, 0)

            accs8 = chunk_phase(g0 + 1, 1, accs8)

            @pl.when(p < NPAIR - 1)
            def _():
                start(g0 + 3, 1)

            return accs8

        accs = list(lax.fori_loop(0, NPAIR, pair_body, tuple(accs)))

        rem_handle.wait()
        rem_on = jnp.where(wid == 0, 1, 0).astype(jnp.int32)
        for g in range(8):
            vb, pb, lb = v_g[g], p_g[g], lab_g[g]
            jb0 = jnp.full((16,), 1, jnp.int32) * REM_START

            def rbody(tt, a, g=g, vb=vb, pb=pb, lb=lb, jb0=jb0):
                jv0 = jb0 + tt * VT
                for jj in range(VT):
                    x = rbuf[tt * VT + jj, pl.ds(16 * g, 16)]
                    thr = jnp.where(jv0 + jj < lb, pb, vb)
                    a = a + jnp.where(x > thr, one, zero)
                return a

            accs[g] = accs[g] + rem_on * (
                lax.fori_loop(0, REM // VT, rbody, zero))

        for g in range(8):
            ostage[pl.ds(16 * g, 16)] = accs[g].astype(jnp.float32)
        pltpu.sync_copy(ostage, out_hbm.at[wid])

    return sc_k, nw


def _tc_reduce(partials):
    def body(x_ref, o_ref):
        ranks = jnp.sum(x_ref[...], axis=0, keepdims=True)
        correct = (ranks < jnp.float32(K)).astype(jnp.float32)
        o_ref[...] = jnp.sum(correct, axis=(0, 1), keepdims=True) * jnp.float32(
            1.0 / ROWS
        )

    return pl.pallas_call(
        body,
        out_shape=jax.ShapeDtypeStruct((1, 1), jnp.float32),
    )(partials)


def kernel(y_probs, y_true_label):
    labels = y_true_label.astype(jnp.int32)
    yt = jnp.transpose(y_probs)
    sc_k, nw = _sc_count_kernel()
    partials = sc_k(yt, labels)
    return _tc_reduce(partials)[0, 0]
